# baseline (speedup 1.0000x reference)
import functools

import jax
import jax.numpy as jnp
from jax import lax
from jax.experimental import pallas as pl
from jax.experimental.pallas import tpu as pltpu
from jax.experimental.pallas import tpu_sc as plsc

_BATCH = 16384
_D = 64
_NW = 32
_BPW = _BATCH // _NW
_L = 16
_CHUNK = 128
_NCHUNK = _BPW // _CHUNK


@functools.partial(
    pl.kernel,
    out_type=jax.ShapeDtypeStruct((_BATCH,), jnp.float32),
    mesh=plsc.VectorSubcoreMesh(core_axis_name="c", subcore_axis_name="s"),
    scratch_types=[
        pltpu.VMEM((_NCHUNK, _CHUNK), jnp.int32),
        pltpu.VMEM((_NCHUNK, _CHUNK), jnp.int32),
        pltpu.VMEM((_BPW, _D), jnp.float32),
        pltpu.VMEM((_BPW, _D), jnp.float32),
        pltpu.VMEM((_BPW,), jnp.float32),
        pltpu.SemaphoreType.DMA,
    ],
    compiler_params=pltpu.CompilerParams(
        needs_layout_passes=False, use_tc_tiling_on_sc=False),
)
def _sc_dot(u_hbm, v_hbm, ue_hbm, ve_hbm, out_hbm,
            u_idx, v_idx, u_rows, v_rows, out_v, sem):
    wid = lax.axis_index("s") * 2 + lax.axis_index("c")
    base = wid * _BPW

    pltpu.sync_copy(u_hbm.at[pl.ds(wid * _NCHUNK, _NCHUNK)], u_idx)
    pltpu.sync_copy(v_hbm.at[pl.ds(wid * _NCHUNK, _NCHUNK)], v_idx)

    copies = []
    for k in range(_NCHUNK):
        copies.append(pltpu.async_copy(
            ue_hbm.at[u_idx.at[k]], u_rows.at[pl.ds(k * _CHUNK, _CHUNK)], sem))
        copies.append(pltpu.async_copy(
            ve_hbm.at[v_idx.at[k]], v_rows.at[pl.ds(k * _CHUNK, _CHUNK)], sem))
    for cp in copies:
        cp.wait()

    lanes = lax.iota(jnp.int32, _L)

    def group(g, carry):
        rows = g * _L + lanes
        accs = [jnp.zeros((_L,), jnp.float32) for _ in range(4)]
        for d in range(_D):
            col = jnp.full((_L,), d, jnp.int32)
            du = plsc.load_gather(u_rows, [rows, col])
            dv = plsc.load_gather(v_rows, [rows, col])
            accs[d % 4] = accs[d % 4] + du * dv
        out_v[pl.ds(g * _L, _L)] = (accs[0] + accs[1]) + (accs[2] + accs[3])
        return carry

    lax.fori_loop(0, _BPW // _L, group, 0)

    pltpu.sync_copy(out_v, out_hbm.at[pl.ds(base, _BPW)])


def kernel(u, v, user_emb, item_emb):
    u2 = u.astype(jnp.int32).reshape(_NW * _NCHUNK, _CHUNK)
    v2 = v.astype(jnp.int32).reshape(_NW * _NCHUNK, _CHUNK)
    return _sc_dot(u2, v2, user_emb, item_emb)


# speedup vs baseline: 3.0316x; 3.0316x over previous
import functools

import jax
import jax.numpy as jnp
from jax import lax
from jax.experimental import pallas as pl
from jax.experimental.pallas import tpu as pltpu
from jax.experimental.pallas import tpu_sc as plsc

_BATCH = 16384
_D = 64
_L = 16
_NT = 7813
_RANGE = 489
_NB = 512
_CAP = 2080
_NS = 8

_params = pltpu.CompilerParams(
    needs_layout_passes=False,
    use_tc_tiling_on_sc=True,
    disable_bounds_checks=True,
)


def _gather_table(idx_hbm, tet_hbm, outc_hbm, s,
                  idx_all, counts, offsets, next_off, ordmap, spos, sidx,
                  ring, stage, sems, sem_out):
    lanes = lax.iota(jnp.int32, _L)
    lo = s * _RANGE
    nrange = jnp.minimum(lo + _RANGE, _NT) - lo

    pltpu.sync_copy(idx_hbm, idx_all)
    for i in range(_NB // _L):
        counts[pl.ds(i * _L, _L)] = jnp.zeros((_L,), jnp.int32)

    def hist(g, carry):
        vec = idx_all[pl.ds(g * _L, _L)]
        rel = (vec >> 7) - lo
        mask = (rel >= 0) & (rel < nrange)
        safe = jnp.where(mask, rel, 0)
        rc, last = plsc.scan_count(safe, mask)
        plsc.addupdate_scatter(counts, [safe], rc, mask=last)
        return carry

    lax.fori_loop(0, _BATCH // _L, hist, 0)

    def excl(i, carry):
        cnt = counts[pl.ds(i * _L, _L)]
        inc = plsc.cumsum(cnt)
        off = inc - cnt + carry
        offsets[pl.ds(i * _L, _L)] = off
        next_off[pl.ds(i * _L, _L)] = off
        return carry + inc[15]

    total = lax.fori_loop(0, _NB // _L, excl, jnp.int32(0))

    def ords(i, carry):
        cnt = counts[pl.ds(i * _L, _L)]
        dm = jnp.where(cnt > 0, 1, 0).astype(jnp.int32)
        dcs = plsc.cumsum(dm) + carry
        ordmap[pl.ds(i * _L, _L)] = dcs - 1
        return dcs[15]

    lax.fori_loop(0, _NB // _L, ords, jnp.int32(0))

    def scat(g, carry):
        vec = idx_all[pl.ds(g * _L, _L)]
        rel = (vec >> 7) - lo
        mask = (rel >= 0) & (rel < nrange)
        safe = jnp.where(mask, rel, 0)
        base = plsc.load_gather(next_off, [safe])
        rc, last = plsc.scan_count(safe, mask)
        dest = jnp.minimum(base + rc - 1, _CAP - 1)
        dsafe = jnp.where(mask, dest, 0)
        pos = g * _L + lanes
        plsc.store_scatter(spos, [dsafe], pos, mask=mask)
        plsc.store_scatter(sidx, [dsafe], vec, mask=mask)
        plsc.addupdate_scatter(next_off, [safe], rc, mask=last)
        return carry

    lax.fori_loop(0, _BATCH // _L, scat, 0)

    ngroups = (total + _L - 1) >> 4

    def wait_slot(o, carry):
        slot = o & (_NS - 1)
        pltpu.make_async_copy(
            tet_hbm.at[:, pl.ds(0, 128)],
            ring.at[pl.ds(0, _D)],
            sems.at[slot],
        ).wait()
        return carry

    def drain_out(k, carry):
        pltpu.make_async_copy(
            outc_hbm.at[pl.ds(0, _D)], stage.at[pl.ds(0, _D)], sem_out
        ).wait()
        return carry

    def body(gg, carry):
        fired, done, nout = carry
        lax.fori_loop(0, nout, drain_out, 0)
        lm = (gg * _L + lanes) < total
        idx16 = sidx[pl.ds(gg * _L, _L)]
        pos16 = spos[pl.ds(gg * _L, _L)]
        rel = jnp.where(lm, (idx16 >> 7) - lo, 0)
        ordv = jnp.where(lm, plsc.load_gather(ordmap, [rel]), 0)
        tc16 = jnp.where(lm, idx16 >> 7, 0)
        dmax = jnp.max(ordv)
        colv = idx16 & 127
        slotbase = (ordv & (_NS - 1)) * _D

        def window(carry_w):
            fr, dn, prev = carry_w
            wlim = jnp.minimum(prev + _NS, dmax)
            for l in range(_L):
                ol = ordv[l]
                tcl = tc16[l]
                cond = (ol > fr) & (ol <= wlim)

                @pl.when(cond)
                def _():
                    pltpu.async_copy(
                        tet_hbm.at[:, pl.ds(tcl * 128, 128)],
                        ring.at[pl.ds((ol & (_NS - 1)) * _D, _D)],
                        sems.at[ol & (_NS - 1)],
                    )

                fr = jnp.where(cond, ol, fr)
            lax.fori_loop(dn + 1, wlim + 1, wait_slot, 0)
            emask = lm & (ordv > prev) & (ordv <= wlim)
            for c in range(_D):
                vals = plsc.load_gather(
                    ring, [slotbase + c, colv], mask=emask)
                plsc.store_scatter(
                    stage, [lanes * _D + c], vals, mask=emask)
            return fr, jnp.maximum(dn, wlim), wlim

        fr, done2, _ = lax.while_loop(
            lambda cw: cw[2] < dmax, window, (fired, done, done - 1))
        lmN = ((gg + 1) * _L + lanes) < total
        idxN = sidx[pl.ds((gg + 1) * _L, _L)]
        relN = jnp.where(lmN, (idxN >> 7) - lo, 0)
        ordN = jnp.where(lmN, plsc.load_gather(ordmap, [relN]), 0)
        tcN = jnp.where(lmN, idxN >> 7, 0)
        cap = dmax + _NS - 1
        for l in range(_L):
            ol = ordN[l]
            tcl = tcN[l]
            cond = (ol > fr) & (ol <= cap)

            @pl.when(cond)
            def _():
                pltpu.async_copy(
                    tet_hbm.at[:, pl.ds(tcl * 128, 128)],
                    ring.at[pl.ds((ol & (_NS - 1)) * _D, _D)],
                    sems.at[ol & (_NS - 1)],
                )

            fr = jnp.where(cond, ol, fr)
        nout2 = jnp.int32(0)
        for l in range(_L):
            posl = pos16[l]
            valid = (gg * _L + l) < total

            @pl.when(valid)
            def _():
                pltpu.async_copy(
                    stage.at[pl.ds(l * _D, _D)],
                    outc_hbm.at[pl.ds(posl * _D, _D)],
                    sem_out,
                )

            nout2 = nout2 + jnp.where(valid, 1, 0).astype(jnp.int32)
        return fr, done2, nout2

    fired, done, nout = lax.fori_loop(
        0, ngroups, body, (jnp.int32(-1), jnp.int32(-1), jnp.int32(0))
    )
    lax.fori_loop(0, nout, drain_out, 0)
    lax.fori_loop(done + 1, fired + 1, wait_slot, 0)


@functools.partial(
    pl.kernel,
    out_type=(
        jax.ShapeDtypeStruct((_BATCH * _D,), jnp.float32),
        jax.ShapeDtypeStruct((_BATCH * _D,), jnp.float32),
    ),
    mesh=plsc.VectorSubcoreMesh(core_axis_name="c", subcore_axis_name="s"),
    scratch_types=[
        pltpu.VMEM((_BATCH,), jnp.int32),
        pltpu.VMEM((_NB,), jnp.int32),
        pltpu.VMEM((_NB,), jnp.int32),
        pltpu.VMEM((_NB,), jnp.int32),
        pltpu.VMEM((_NB,), jnp.int32),
        pltpu.VMEM((_CAP,), jnp.int32),
        pltpu.VMEM((_CAP,), jnp.int32),
        pltpu.VMEM((_NS * _D, 128), jnp.float32),
        pltpu.VMEM((_L * _D,), jnp.float32),
        pltpu.SemaphoreType.DMA((_NS,)),
        pltpu.SemaphoreType.DMA,
    ],
    compiler_params=_params,
)
def _phase_a(u_hbm, v_hbm, uet_hbm, vet_hbm, ucols_hbm, vcols_hbm, *scratch):
    c = lax.axis_index("c")
    s = lax.axis_index("s")

    @pl.when(c == 0)
    def _():
        _gather_table(u_hbm, uet_hbm, ucols_hbm, s, *scratch)

    @pl.when(c == 1)
    def _():
        _gather_table(v_hbm, vet_hbm, vcols_hbm, s, *scratch)


@functools.partial(
    pl.kernel,
    out_type=jax.ShapeDtypeStruct((_BATCH,), jnp.float32),
    mesh=plsc.VectorSubcoreMesh(core_axis_name="c", subcore_axis_name="s"),
    scratch_types=[
        pltpu.VMEM((512 * _D,), jnp.float32),
        pltpu.VMEM((512 * _D,), jnp.float32),
        pltpu.VMEM((512,), jnp.float32),
    ],
    compiler_params=_params,
)
def _phase_b(ucols_hbm, vcols_hbm, out_hbm, ubuf, vbuf, out_v):
    wid = lax.axis_index("s") * 2 + lax.axis_index("c")
    base = wid * 512
    lanes = lax.iota(jnp.int32, _L)
    pltpu.sync_copy(ucols_hbm.at[pl.ds(base * _D, 512 * _D)], ubuf)
    pltpu.sync_copy(vcols_hbm.at[pl.ds(base * _D, 512 * _D)], vbuf)

    def group(j, carry):
        flat0 = (j * _L + lanes) * _D
        accs = [jnp.zeros((_L,), jnp.float32) for _ in range(4)]
        for d in range(_D):
            du = plsc.load_gather(ubuf, [flat0 + d])
            dv = plsc.load_gather(vbuf, [flat0 + d])
            accs[d % 4] = accs[d % 4] + du * dv
        out_v[pl.ds(j * _L, _L)] = (accs[0] + accs[1]) + (accs[2] + accs[3])
        return carry

    lax.fori_loop(0, 512 // _L, group, 0)
    pltpu.sync_copy(out_v, out_hbm.at[pl.ds(base, 512)])


def kernel(u, v, user_emb, item_emb):
    ucols, vcols = _phase_a(u.astype(jnp.int32), v.astype(jnp.int32),
                            user_emb.T, item_emb.T)
    return _phase_b(ucols, vcols)


# speedup vs baseline: 3.5233x; 1.1622x over previous
import functools

import jax
import jax.numpy as jnp
from jax import lax
from jax.experimental import pallas as pl
from jax.experimental.pallas import tpu as pltpu
from jax.experimental.pallas import tpu_sc as plsc

_BATCH = 16384
_D = 64
_L = 16
_NT = 7813
_RANGE = 489
_NB = 512
_CAP = 2144
_NS = 12
_W = 8

_params = pltpu.CompilerParams(
    needs_layout_passes=False,
    use_tc_tiling_on_sc=True,
    disable_bounds_checks=True,
)


def _gather_table(idx_hbm, tet_hbm, outc_hbm, s,
                  idx_all, counts, offsets, next_off, ordmap, spos, sidx,
                  ring, stage, sems, sem_out):
    lanes = lax.iota(jnp.int32, _L)
    lo = s * _RANGE
    nrange = jnp.minimum(lo + _RANGE, _NT) - lo

    pltpu.sync_copy(idx_hbm, idx_all)
    for i in range(_NB // _L):
        counts[pl.ds(i * _L, _L)] = jnp.zeros((_L,), jnp.int32)

    def hist(g, carry):
        vec = idx_all[pl.ds(g * _L, _L)]
        rel = (vec >> 7) - lo
        mask = (rel >= 0) & (rel < nrange)
        safe = jnp.where(mask, rel, 0)
        rc, last = plsc.scan_count(safe, mask)
        plsc.addupdate_scatter(counts, [safe], rc, mask=last)
        return carry

    lax.fori_loop(0, _BATCH // _L, hist, 0)

    def excl(i, carry):
        cnt = counts[pl.ds(i * _L, _L)]
        inc = plsc.cumsum(cnt)
        off = inc - cnt + carry
        offsets[pl.ds(i * _L, _L)] = off
        next_off[pl.ds(i * _L, _L)] = off
        return carry + inc[15]

    total = lax.fori_loop(0, _NB // _L, excl, jnp.int32(0))

    def ords(i, carry):
        cnt = counts[pl.ds(i * _L, _L)]
        dm = jnp.where(cnt > 0, 1, 0).astype(jnp.int32)
        dcs = plsc.cumsum(dm) + carry
        ordmap[pl.ds(i * _L, _L)] = dcs - 1
        return dcs[15]

    lax.fori_loop(0, _NB // _L, ords, jnp.int32(0))

    def scat(g, carry):
        vec = idx_all[pl.ds(g * _L, _L)]
        rel = (vec >> 7) - lo
        mask = (rel >= 0) & (rel < nrange)
        safe = jnp.where(mask, rel, 0)
        base = plsc.load_gather(next_off, [safe])
        rc, last = plsc.scan_count(safe, mask)
        dest = jnp.minimum(base + rc - 1, _CAP - 1)
        dsafe = jnp.where(mask, dest, 0)
        pos = g * _L + lanes
        plsc.store_scatter(spos, [dsafe], pos, mask=mask)
        plsc.store_scatter(sidx, [dsafe], vec, mask=mask)
        plsc.addupdate_scatter(next_off, [safe], rc, mask=last)
        return carry

    lax.fori_loop(0, _BATCH // _L, scat, 0)

    ngroups = (total + _L - 1) >> 4

    def wait_slot(o, carry):
        slot = lax.rem(o, _NS)
        pltpu.make_async_copy(
            tet_hbm.at[:, pl.ds(0, 128)],
            ring.at[pl.ds(0, _D)],
            sems.at[slot],
        ).wait()
        return carry

    def drain_out(k, carry):
        pltpu.make_async_copy(
            outc_hbm.at[pl.ds(0, _D)], stage.at[pl.ds(0, _D)], sem_out
        ).wait()
        return carry

    def body(gg, carry):
        fired, done, nout = carry
        lax.fori_loop(0, nout, drain_out, 0)
        lm = (gg * _L + lanes) < total
        idx16 = sidx[pl.ds(gg * _L, _L)]
        pos16 = spos[pl.ds(gg * _L, _L)]
        rel = jnp.where(lm, (idx16 >> 7) - lo, 0)
        ordv = jnp.where(lm, plsc.load_gather(ordmap, [rel]), 0)
        tc16 = jnp.where(lm, idx16 >> 7, 0)
        dmax = jnp.max(ordv)
        colv = idx16 & 127
        slotbase = lax.rem(ordv, _NS) * _D

        def window(carry_w):
            fr, dn, prev = carry_w
            wlim = jnp.minimum(prev + _W, dmax)
            fcap = jnp.minimum(prev + _NS, dmax)
            for l in range(_L):
                ol = ordv[l]
                tcl = tc16[l]
                cond = (ol > fr) & (ol <= fcap)

                @pl.when(cond)
                def _():
                    pltpu.async_copy(
                        tet_hbm.at[:, pl.ds(tcl * 128, 128)],
                        ring.at[pl.ds(lax.rem(ol, _NS) * _D, _D)],
                        sems.at[lax.rem(ol, _NS)],
                    )

                fr = jnp.where(cond, ol, fr)
            lax.fori_loop(dn + 1, wlim + 1, wait_slot, 0)
            emask = lm & (ordv > prev) & (ordv <= wlim)
            for c in range(_D):
                vals = plsc.load_gather(
                    ring, [slotbase + c, colv], mask=emask)
                plsc.store_scatter(
                    stage, [lanes * _D + c], vals, mask=emask)
            return fr, jnp.maximum(dn, wlim), wlim

        fr, done2, _ = lax.while_loop(
            lambda cw: cw[2] < dmax, window, (fired, done, done - 1))
        cap = dmax + _NS - 1
        for ahead in (1, 2):
            lmN = ((gg + ahead) * _L + lanes) < total
            idxN = sidx[pl.ds((gg + ahead) * _L, _L)]
            relN = jnp.where(lmN, (idxN >> 7) - lo, 0)
            ordN = jnp.where(lmN, plsc.load_gather(ordmap, [relN]), 0)
            tcN = jnp.where(lmN, idxN >> 7, 0)
            for l in range(_L):
                ol = ordN[l]
                tcl = tcN[l]
                cond = (ol > fr) & (ol <= cap)

                @pl.when(cond)
                def _():
                    pltpu.async_copy(
                        tet_hbm.at[:, pl.ds(tcl * 128, 128)],
                        ring.at[pl.ds(lax.rem(ol, _NS) * _D, _D)],
                        sems.at[lax.rem(ol, _NS)],
                    )

                fr = jnp.where(cond, ol, fr)
        nout2 = jnp.int32(0)
        for l in range(_L):
            posl = pos16[l]
            valid = (gg * _L + l) < total

            @pl.when(valid)
            def _():
                pltpu.async_copy(
                    stage.at[pl.ds(l * _D, _D)],
                    outc_hbm.at[pl.ds(posl * _D, _D)],
                    sem_out,
                )

            nout2 = nout2 + jnp.where(valid, 1, 0).astype(jnp.int32)
        return fr, done2, nout2

    fired, done, nout = lax.fori_loop(
        0, ngroups, body, (jnp.int32(-1), jnp.int32(-1), jnp.int32(0))
    )
    lax.fori_loop(0, nout, drain_out, 0)
    lax.fori_loop(done + 1, fired + 1, wait_slot, 0)


@functools.partial(
    pl.kernel,
    out_type=(
        jax.ShapeDtypeStruct((_BATCH * _D,), jnp.float32),
        jax.ShapeDtypeStruct((_BATCH * _D,), jnp.float32),
    ),
    mesh=plsc.VectorSubcoreMesh(core_axis_name="c", subcore_axis_name="s"),
    scratch_types=[
        pltpu.VMEM((_BATCH,), jnp.int32),
        pltpu.VMEM((_NB,), jnp.int32),
        pltpu.VMEM((_NB,), jnp.int32),
        pltpu.VMEM((_NB,), jnp.int32),
        pltpu.VMEM((_NB,), jnp.int32),
        pltpu.VMEM((_CAP,), jnp.int32),
        pltpu.VMEM((_CAP,), jnp.int32),
        pltpu.VMEM((_NS * _D, 128), jnp.float32),
        pltpu.VMEM((_L * _D,), jnp.float32),
        pltpu.SemaphoreType.DMA((_NS,)),
        pltpu.SemaphoreType.DMA,
    ],
    compiler_params=_params,
)
def _phase_a(u_hbm, v_hbm, uet_hbm, vet_hbm, ucols_hbm, vcols_hbm, *scratch):
    c = lax.axis_index("c")
    s = lax.axis_index("s")

    @pl.when(c == 0)
    def _():
        _gather_table(u_hbm, uet_hbm, ucols_hbm, s, *scratch)

    @pl.when(c == 1)
    def _():
        _gather_table(v_hbm, vet_hbm, vcols_hbm, s, *scratch)


@functools.partial(
    pl.kernel,
    out_type=jax.ShapeDtypeStruct((_BATCH,), jnp.float32),
    mesh=plsc.VectorSubcoreMesh(core_axis_name="c", subcore_axis_name="s"),
    scratch_types=[
        pltpu.VMEM((512 * _D,), jnp.float32),
        pltpu.VMEM((512 * _D,), jnp.float32),
        pltpu.VMEM((512,), jnp.float32),
    ],
    compiler_params=_params,
)
def _phase_b(ucols_hbm, vcols_hbm, out_hbm, ubuf, vbuf, out_v):
    wid = lax.axis_index("s") * 2 + lax.axis_index("c")
    base = wid * 512
    lanes = lax.iota(jnp.int32, _L)
    pltpu.sync_copy(ucols_hbm.at[pl.ds(base * _D, 512 * _D)], ubuf)
    pltpu.sync_copy(vcols_hbm.at[pl.ds(base * _D, 512 * _D)], vbuf)

    def group(j, carry):
        flat0 = (j * _L + lanes) * _D
        accs = [jnp.zeros((_L,), jnp.float32) for _ in range(4)]
        for d in range(_D):
            du = plsc.load_gather(ubuf, [flat0 + d])
            dv = plsc.load_gather(vbuf, [flat0 + d])
            accs[d % 4] = accs[d % 4] + du * dv
        out_v[pl.ds(j * _L, _L)] = (accs[0] + accs[1]) + (accs[2] + accs[3])
        return carry

    lax.fori_loop(0, 512 // _L, group, 0)
    pltpu.sync_copy(out_v, out_hbm.at[pl.ds(base, 512)])


def kernel(u, v, user_emb, item_emb):
    ucols, vcols = _phase_a(u.astype(jnp.int32), v.astype(jnp.int32),
                            user_emb.T, item_emb.T)
    return _phase_b(ucols, vcols)


# speedup vs baseline: 3.7194x; 1.0557x over previous
import functools

import jax
import jax.numpy as jnp
from jax import lax
from jax.experimental import pallas as pl
from jax.experimental.pallas import tpu as pltpu
from jax.experimental.pallas import tpu_sc as plsc

_BATCH = 16384
_D = 64
_L = 16
_NT = 7813
_RANGE = 489
_NB = 512
_CAP = 2144
_NS = 12
_W = 8

_params = pltpu.CompilerParams(
    needs_layout_passes=False,
    use_tc_tiling_on_sc=True,
    disable_bounds_checks=True,
)


def _gather_table(idx_hbm, tet_hbm, outc_hbm, s,
                  idx_all, counts, offsets, next_off, ordmap, dist_tc,
                  spos, sidx, ring, stage, sems, sem_out):
    lanes = lax.iota(jnp.int32, _L)
    lo = s * _RANGE
    nrange = jnp.minimum(lo + _RANGE, _NT) - lo

    pltpu.sync_copy(idx_hbm, idx_all)
    for i in range(_NB // _L):
        counts[pl.ds(i * _L, _L)] = jnp.zeros((_L,), jnp.int32)

    def hist(g, carry):
        vec = idx_all[pl.ds(g * _L, _L)]
        rel = (vec >> 7) - lo
        mask = (rel >= 0) & (rel < nrange)
        safe = jnp.where(mask, rel, 0)
        rc, last = plsc.scan_count(safe, mask)
        plsc.addupdate_scatter(counts, [safe], rc, mask=last)
        return carry

    lax.fori_loop(0, _BATCH // _L, hist, 0)

    def excl(i, carry):
        cnt = counts[pl.ds(i * _L, _L)]
        inc = plsc.cumsum(cnt)
        off = inc - cnt + carry
        offsets[pl.ds(i * _L, _L)] = off
        next_off[pl.ds(i * _L, _L)] = off
        return carry + inc[15]

    total = lax.fori_loop(0, _NB // _L, excl, jnp.int32(0))

    def ords(i, carry):
        cnt = counts[pl.ds(i * _L, _L)]
        m = cnt > 0
        dm = jnp.where(m, 1, 0).astype(jnp.int32)
        dcs = plsc.cumsum(dm) + carry
        ov = dcs - 1
        ordmap[pl.ds(i * _L, _L)] = ov
        tcv = lo + i * _L + lanes
        plsc.store_scatter(dist_tc, [jnp.where(m, ov, 0)], tcv, mask=m)
        return dcs[15]

    nd = lax.fori_loop(0, _NB // _L, ords, jnp.int32(0))

    def scat(g, carry):
        vec = idx_all[pl.ds(g * _L, _L)]
        rel = (vec >> 7) - lo
        mask = (rel >= 0) & (rel < nrange)
        safe = jnp.where(mask, rel, 0)
        base = plsc.load_gather(next_off, [safe])
        rc, last = plsc.scan_count(safe, mask)
        dest = jnp.minimum(base + rc - 1, _CAP - 1)
        dsafe = jnp.where(mask, dest, 0)
        pos = g * _L + lanes
        plsc.store_scatter(spos, [dsafe], pos, mask=mask)
        plsc.store_scatter(sidx, [dsafe], vec, mask=mask)
        plsc.addupdate_scatter(next_off, [safe], rc, mask=last)
        return carry

    lax.fori_loop(0, _BATCH // _L, scat, 0)

    ngroups = (total + _L - 1) >> 4

    def wait_slot(o, carry):
        slot = lax.rem(o, _NS)
        pltpu.make_async_copy(
            tet_hbm.at[:, pl.ds(0, 128)],
            ring.at[pl.ds(0, _D)],
            sems.at[slot],
        ).wait()
        return carry

    def sread(ref, d):
        va = ref[pl.ds(pl.multiple_of((d >> 4) * _L, _L), _L)]
        lane = jnp.full((_L,), d & (_L - 1), jnp.int32)
        g = jax.lax.gather(
            va, lane[:, None],
            jax.lax.GatherDimensionNumbers(
                offset_dims=(), collapsed_slice_dims=(0,),
                start_index_map=(0,)),
            slice_sizes=(1,),
            mode=jax.lax.GatherScatterMode.PROMISE_IN_BOUNDS)
        return g[0]

    def fire_one(o, carry):
        tcl = sread(dist_tc, o)
        slot = lax.rem(o, _NS)
        pltpu.async_copy(
            tet_hbm.at[:, pl.ds(tcl * 128, 128)],
            ring.at[pl.ds(slot * _D, _D)],
            sems.at[slot],
        )
        return carry

    def drain_out(k, carry):
        pltpu.make_async_copy(
            outc_hbm.at[pl.ds(0, _D)], stage.at[pl.ds(0, _D)], sem_out
        ).wait()
        return carry

    def body(gg, carry):
        fired, done, nout = carry
        lax.fori_loop(0, nout, drain_out, 0)
        lm = (gg * _L + lanes) < total
        idx16 = sidx[pl.ds(gg * _L, _L)]
        pos16 = spos[pl.ds(gg * _L, _L)]
        rel = jnp.where(lm, (idx16 >> 7) - lo, 0)
        ordv = jnp.where(lm, plsc.load_gather(ordmap, [rel]), 0)
        tc16 = jnp.where(lm, idx16 >> 7, 0)
        dmax = jnp.max(ordv)
        colv = idx16 & 127
        slotbase = lax.rem(ordv, _NS) * _D

        def window(carry_w):
            fr, dn, prev = carry_w
            wlim = jnp.minimum(prev + _W, dmax)
            fcap = jnp.minimum(prev + _NS - 1, nd - 1)
            lax.fori_loop(fr + 1, fcap + 1, fire_one, 0)
            fr = jnp.maximum(fr, fcap)
            lax.fori_loop(dn + 1, wlim + 1, wait_slot, 0)
            emask = lm & (ordv > prev) & (ordv <= wlim)
            for c in range(_D):
                vals = plsc.load_gather(
                    ring, [slotbase + c, colv], mask=emask)
                plsc.store_scatter(
                    stage, [lanes * _D + c], vals, mask=emask)
            return fr, jnp.maximum(dn, wlim), wlim

        fr, done2, _ = lax.while_loop(
            lambda cw: cw[2] < dmax, window, (fired, done, done - 1))
        pcap = jnp.minimum(dmax + _NS - 1, nd - 1)
        lax.fori_loop(fr + 1, pcap + 1, fire_one, 0)
        fr = jnp.maximum(fr, pcap)
        nout2 = jnp.int32(0)
        for l in range(_L):
            posl = pos16[l]
            valid = (gg * _L + l) < total

            @pl.when(valid)
            def _():
                pltpu.async_copy(
                    stage.at[pl.ds(l * _D, _D)],
                    outc_hbm.at[pl.ds(posl * _D, _D)],
                    sem_out,
                )

            nout2 = nout2 + jnp.where(valid, 1, 0).astype(jnp.int32)
        return fr, done2, nout2

    fired, done, nout = lax.fori_loop(
        0, ngroups, body, (jnp.int32(-1), jnp.int32(-1), jnp.int32(0))
    )
    lax.fori_loop(0, nout, drain_out, 0)
    lax.fori_loop(done + 1, fired + 1, wait_slot, 0)


@functools.partial(
    pl.kernel,
    out_type=(
        jax.ShapeDtypeStruct((_BATCH * _D,), jnp.float32),
        jax.ShapeDtypeStruct((_BATCH * _D,), jnp.float32),
    ),
    mesh=plsc.VectorSubcoreMesh(core_axis_name="c", subcore_axis_name="s"),
    scratch_types=[
        pltpu.VMEM((_BATCH,), jnp.int32),
        pltpu.VMEM((_NB,), jnp.int32),
        pltpu.VMEM((_NB,), jnp.int32),
        pltpu.VMEM((_NB,), jnp.int32),
        pltpu.VMEM((_NB,), jnp.int32),
        pltpu.VMEM((_NB,), jnp.int32),
        pltpu.VMEM((_CAP,), jnp.int32),
        pltpu.VMEM((_CAP,), jnp.int32),
        pltpu.VMEM((_NS * _D, 128), jnp.float32),
        pltpu.VMEM((_L * _D,), jnp.float32),
        pltpu.SemaphoreType.DMA((_NS,)),
        pltpu.SemaphoreType.DMA,
    ],
    compiler_params=_params,
)
def _phase_a(u_hbm, v_hbm, uet_hbm, vet_hbm, ucols_hbm, vcols_hbm, *scratch):
    c = lax.axis_index("c")
    s = lax.axis_index("s")

    @pl.when(c == 0)
    def _():
        _gather_table(u_hbm, uet_hbm, ucols_hbm, s, *scratch)

    @pl.when(c == 1)
    def _():
        _gather_table(v_hbm, vet_hbm, vcols_hbm, s, *scratch)


@functools.partial(
    pl.kernel,
    out_type=jax.ShapeDtypeStruct((_BATCH,), jnp.float32),
    mesh=plsc.VectorSubcoreMesh(core_axis_name="c", subcore_axis_name="s"),
    scratch_types=[
        pltpu.VMEM((512 * _D,), jnp.float32),
        pltpu.VMEM((512 * _D,), jnp.float32),
        pltpu.VMEM((512,), jnp.float32),
    ],
    compiler_params=_params,
)
def _phase_b(ucols_hbm, vcols_hbm, out_hbm, ubuf, vbuf, out_v):
    wid = lax.axis_index("s") * 2 + lax.axis_index("c")
    base = wid * 512
    lanes = lax.iota(jnp.int32, _L)
    pltpu.sync_copy(ucols_hbm.at[pl.ds(base * _D, 512 * _D)], ubuf)
    pltpu.sync_copy(vcols_hbm.at[pl.ds(base * _D, 512 * _D)], vbuf)

    def group(j, carry):
        flat0 = (j * _L + lanes) * _D
        accs = [jnp.zeros((_L,), jnp.float32) for _ in range(4)]
        for d in range(_D):
            du = plsc.load_gather(ubuf, [flat0 + d])
            dv = plsc.load_gather(vbuf, [flat0 + d])
            accs[d % 4] = accs[d % 4] + du * dv
        out_v[pl.ds(j * _L, _L)] = (accs[0] + accs[1]) + (accs[2] + accs[3])
        return carry

    lax.fori_loop(0, 512 // _L, group, 0)
    pltpu.sync_copy(out_v, out_hbm.at[pl.ds(base, 512)])


def kernel(u, v, user_emb, item_emb):
    ucols, vcols = _phase_a(u.astype(jnp.int32), v.astype(jnp.int32),
                            user_emb.T, item_emb.T)
    return _phase_b(ucols, vcols)
